# baseline (speedup 1.0000x reference)
import functools

import jax
import jax.numpy as jnp
import numpy as np
from jax.experimental import pallas as pl

U32 = jnp.uint32

_RATIO = 0.3
_ZERO_RATIO = 0.8
_RANDOM_RATIO = 0.1
_MAX_TIMESPAN = 8
_EXPAND_PROB = 0.5

_key = jax.random.key(42)
_k1, _k2, _k3, _k4, _k5, _k6 = jax.random.split(_key, 6)
_EXPAND = bool(jax.random.bernoulli(_k1, _EXPAND_PROB))
_TIMESPAN = int(jax.random.randint(_k2, (), 1, _MAX_TIMESPAN + 1)) if _EXPAND else 1
_MASK_RATIO = _RATIO / _TIMESPAN
_KD3 = tuple(int(v) for v in np.asarray(jax.random.key_data(_k3)))
_KD4 = tuple(int(v) for v in np.asarray(jax.random.key_data(_k4)))
_KD5 = tuple(int(v) for v in np.asarray(jax.random.key_data(_k5)))
_KD6 = tuple(int(v) for v in np.asarray(jax.random.key_data(_k6)))
_PAD_LOW = (_TIMESPAN - 1) // 2
_WIN = tuple(range(-_PAD_LOW, _TIMESPAN - _PAD_LOW))


def _rotl(x, d):
    return (x << U32(d)) | (x >> U32(32 - d))


def _tf_bits(kd, idx):
    ks0 = U32(kd[0])
    ks1 = U32(kd[1])
    ks2 = ks0 ^ ks1 ^ U32(0x1BD11BDA)
    ks = (ks0, ks1, ks2)
    rot = ((13, 15, 26, 6), (17, 29, 16, 24))
    x0 = jnp.full(idx.shape, ks0, U32)
    x1 = idx + ks1
    for i in range(5):
        for r in rot[i % 2]:
            x0 = x0 + x1
            x1 = _rotl(x1, r)
            x1 = x1 ^ x0
        x0 = x0 + ks[(i + 1) % 3]
        x1 = x1 + ks[(i + 2) % 3] + U32(i + 1)
    return x0 ^ x1


def _tf_uniform(kd, idx):
    bits = _tf_bits(kd, idx)
    f = jax.lax.bitcast_convert_type((bits >> U32(9)) | U32(0x3F800000), jnp.float32)
    return f - 1.0


def _mask_kernel(out_ref, *, B, T):
    b = jax.lax.broadcasted_iota(jnp.int32, (B, T), 0)
    t = jax.lax.broadcasted_iota(jnp.int32, (B, T), 1)
    base = (b * T + t).astype(U32)
    acc = jnp.zeros((B, T), jnp.bool_)
    for d in _WIN:
        u = _tf_uniform(_KD3, base + U32(np.uint32(d)))
        hit = u < jnp.float32(_MASK_RATIO)
        if d != 0:
            valid = (t + d >= 0) & (t + d < T)
            hit = hit & valid
        acc = acc | hit
    out_ref[...] = acc.astype(jnp.float32)


def _flags_max_kernel(x_ref, m_ref, flags_ref, gmax_ref, *, T, N):
    b = pl.program_id(0)
    x = x_ref[0]
    m = m_ref[0] > 0.0
    t = jax.lax.broadcasted_iota(jnp.int32, (T, N), 0)
    n = jax.lax.broadcasted_iota(jnp.int32, (T, N), 1)
    idx = (b * (T * N) + t * N + n).astype(U32)
    u4 = _tf_uniform(_KD4, idx)
    z = (u4 < jnp.float32(_ZERO_RATIO)) & m
    zeroed = jnp.where(z, jnp.float32(0.0), x)
    pmax = jnp.max(zeroed)[None, None]
    flags_ref[0] = z.astype(jnp.int8)

    @pl.when(b == 0)
    def _init():
        gmax_ref[...] = pmax

    @pl.when(b != 0)
    def _acc():
        gmax_ref[...] = jnp.maximum(gmax_ref[...], pmax)


def _apply_kernel(x_ref, m_ref, flags_ref, gmax_ref, out_ref, tmask_ref, *, T, N):
    b = pl.program_id(0)
    x = x_ref[0]
    m = m_ref[0] > 0.0
    z = flags_ref[0] != 0
    gmax = gmax_ref[...]
    t = jax.lax.broadcasted_iota(jnp.int32, (T, N), 0)
    n = jax.lax.broadcasted_iota(jnp.int32, (T, N), 1)
    idx = (b * (T * N) + t * N + n).astype(U32)
    u5 = _tf_uniform(_KD5, idx)
    ridx = (u5 < jnp.float32(_RANDOM_RATIO)) & m & (~z)
    u6 = _tf_uniform(_KD6, idx)
    zeroed = jnp.where(z, jnp.float32(0.0), x)
    out_ref[0] = jnp.where(ridx, gmax * u6, zeroed)
    tmask_ref[0] = jnp.broadcast_to(m, (T, N)).astype(jnp.int32)


@jax.jit
def kernel(spikes):
    B, T, N = spikes.shape

    mask2d = pl.pallas_call(
        functools.partial(_mask_kernel, B=B, T=T),
        out_shape=jax.ShapeDtypeStruct((B, T), jnp.float32),
    )()
    maskc = mask2d[:, :, None]

    flags, gmax = pl.pallas_call(
        functools.partial(_flags_max_kernel, T=T, N=N),
        grid=(B,),
        in_specs=[
            pl.BlockSpec((1, T, N), lambda b: (b, 0, 0)),
            pl.BlockSpec((1, T, 1), lambda b: (b, 0, 0)),
        ],
        out_specs=[
            pl.BlockSpec((1, T, N), lambda b: (b, 0, 0)),
            pl.BlockSpec((1, 1), lambda b: (0, 0)),
        ],
        out_shape=[
            jax.ShapeDtypeStruct((B, T, N), jnp.int8),
            jax.ShapeDtypeStruct((1, 1), jnp.float32),
        ],
    )(spikes, maskc)

    out, tmask = pl.pallas_call(
        functools.partial(_apply_kernel, T=T, N=N),
        grid=(B,),
        in_specs=[
            pl.BlockSpec((1, T, N), lambda b: (b, 0, 0)),
            pl.BlockSpec((1, T, 1), lambda b: (b, 0, 0)),
            pl.BlockSpec((1, T, N), lambda b: (b, 0, 0)),
            pl.BlockSpec((1, 1), lambda b: (0, 0)),
        ],
        out_specs=[
            pl.BlockSpec((1, T, N), lambda b: (b, 0, 0)),
            pl.BlockSpec((1, T, N), lambda b: (b, 0, 0)),
        ],
        out_shape=[
            jax.ShapeDtypeStruct((B, T, N), jnp.float32),
            jax.ShapeDtypeStruct((B, T, N), jnp.int32),
        ],
    )(spikes, maskc, flags, gmax)

    return (out, tmask.astype(jnp.int64))
